# baseline (speedup 1.0000x reference)
import functools

import jax
import jax.numpy as jnp
from jax import lax
from jax.experimental import pallas as pl
from jax.experimental.pallas import tpu as pltpu
from jax.experimental.pallas import tpu_sc as plsc

B = 4096
L = 50
DIM = 64
N = B * L

NC = 2
NS = 16
NW = NC * NS
ROWS_PER_W = N // NW
CHUNK = 800
NCHUNK = ROWS_PER_W // CHUNK
NBUF = 2


def _sc_gather(idx_hbm, table_hbm, out_hbm, idx_v, rows_v, gsem, osem):
  wid = lax.axis_index("s") * NC + lax.axis_index("c")
  base = wid * ROWS_PER_W
  pltpu.sync_copy(idx_hbm.at[pl.ds(base, ROWS_PER_W)], idx_v)

  def gather_start(g, buf):
    off = g * CHUNK
    pltpu.async_copy(
        table_hbm.at[idx_v.at[pl.ds(off, CHUNK)]], rows_v.at[buf], gsem
    )

  def out_start(g, buf):
    off = g * CHUNK
    pltpu.async_copy(rows_v.at[buf], out_hbm.at[pl.ds(base + off, CHUNK)], osem)

  def gather_wait(buf):
    pltpu.make_async_copy(
        table_hbm.at[idx_v.at[pl.ds(0, CHUNK)]], rows_v.at[buf], gsem
    ).wait()

  def out_wait(buf):
    pltpu.make_async_copy(
        rows_v.at[buf], out_hbm.at[pl.ds(base, CHUNK)], osem
    ).wait()

  gather_start(0, 0)
  for g in range(NCHUNK):
    buf = g % NBUF
    nbuf = (g + 1) % NBUF
    gather_wait(buf)
    if g + 1 < NCHUNK:
      if g >= 1:
        out_wait(nbuf)
      gather_start(g + 1, nbuf)
    out_start(g, buf)
  out_wait(0)
  out_wait(1)


@jax.jit
def _embedding(idx_flat, table):
  mesh = plsc.VectorSubcoreMesh(core_axis_name="c", subcore_axis_name="s")
  f = pl.kernel(
      _sc_gather,
      out_type=jax.ShapeDtypeStruct((N, DIM), jnp.float32),
      mesh=mesh,
      scratch_types=[
          pltpu.VMEM((ROWS_PER_W,), jnp.int32),
          pltpu.VMEM((NBUF, CHUNK, DIM), jnp.float32),
          pltpu.SemaphoreType.DMA,
          pltpu.SemaphoreType.DMA,
      ],
      compiler_params=pltpu.CompilerParams(use_tc_tiling_on_sc=False),
  )
  return f(idx_flat, table)


def kernel(input, table):
  idx_flat = input.reshape(N).astype(jnp.int32)
  out = _embedding(idx_flat, table)
  return out.reshape(B, L, DIM)


# speedup vs baseline: 1.0159x; 1.0159x over previous
import functools

import jax
import jax.numpy as jnp
from jax import lax
from jax.experimental import pallas as pl
from jax.experimental.pallas import tpu as pltpu
from jax.experimental.pallas import tpu_sc as plsc

B = 4096
L = 50
DIM = 64
N = B * L

NC = 2
NS = 16
NW = NC * NS
ROWS_PER_W = N // NW
CHUNK = 800
NCHUNK = ROWS_PER_W // CHUNK
NBUF = 2


def _sc_gather(idx_hbm, table_hbm, out_hbm, idx_v, rows_v, gsem, osem):
  wid = lax.axis_index("s") * NC + lax.axis_index("c")
  base = wid * ROWS_PER_W
  pltpu.sync_copy(idx_hbm.at[pl.ds(base, ROWS_PER_W)], idx_v)

  def gather_start(g, buf):
    off = g * CHUNK
    pltpu.async_copy(
        table_hbm.at[idx_v.at[pl.ds(off, CHUNK)]], rows_v.at[buf], gsem
    )

  def out_start(g, buf):
    off = g * CHUNK
    pltpu.async_copy(rows_v.at[buf], out_hbm.at[pl.ds(base + off, CHUNK)], osem)

  def gather_wait(buf):
    pltpu.make_async_copy(
        table_hbm.at[idx_v.at[pl.ds(0, CHUNK)]], rows_v.at[buf], gsem
    ).wait()

  def out_wait(buf):
    pltpu.make_async_copy(
        rows_v.at[buf], out_hbm.at[pl.ds(base, CHUNK)], osem
    ).wait()

  gather_start(0, 0)
  for g in range(NCHUNK):
    buf = g % NBUF
    nbuf = (g + 1) % NBUF
    gather_wait(buf)
    if g + 1 < NCHUNK:
      if g >= 1:
        out_wait(nbuf)
      gather_start(g + 1, nbuf)
    out_start(g, buf)
  out_wait(0)
  out_wait(1)


@jax.jit
def _embedding(idx_flat, table):
  mesh = plsc.VectorSubcoreMesh(core_axis_name="c", subcore_axis_name="s")
  f = pl.kernel(
      _sc_gather,
      out_type=jax.ShapeDtypeStruct((N, DIM), jnp.float32),
      mesh=mesh,
      scratch_types=[
          pltpu.VMEM((ROWS_PER_W,), jnp.int32),
          pltpu.VMEM((NBUF, CHUNK, DIM), jnp.float32),
          pltpu.SemaphoreType.DMA,
          pltpu.SemaphoreType.DMA,
      ],
      compiler_params=pltpu.CompilerParams(use_tc_tiling_on_sc=False),
  )
  return f(idx_flat, table)


def kernel(input, table):
  idx_flat = input.T.reshape(N).astype(jnp.int32)
  out = _embedding(idx_flat, table)
  return out.reshape(L, B, DIM).transpose(1, 0, 2)
